# baseline (speedup 1.0000x reference)
import functools

import jax
import jax.numpy as jnp
from jax import lax
from jax.experimental import pallas as pl
from jax.experimental.pallas import tpu as pltpu
from jax.experimental.pallas import tpu_sc as plsc

_B = 16384
_C = 84
_L = 3
_NW = 32
_ROWS = _L * _C


def _sc_body(s_hbm, mf_hbm, out_hbm, row_v, fill_v, mask_v, sem):
    nc = 2
    wid = lax.axis_index("s") * nc + lax.axis_index("c")

    pltpu.sync_copy(mf_hbm, mask_v)

    neg = jnp.full((16,), -jnp.inf, dtype=jnp.float32)

    def _fill(i, carry):
        fill_v[pl.ds(i * 16, 16)] = neg
        return carry

    lax.fori_loop(0, _B // 16, _fill, 0)

    def _row(k, carry):
        r = wid + k * _NW

        @pl.when(r < _ROWS)
        def _():
            l = r // _C
            c = r - l * _C
            flag = mask_v[pl.ds(r, 16)][0]

            @pl.when(flag == 0)
            def _():
                pltpu.sync_copy(s_hbm.at[c], row_v)
                pltpu.sync_copy(row_v, out_hbm.at[l, c])

            @pl.when(flag != 0)
            def _():
                pltpu.sync_copy(fill_v, out_hbm.at[l, c])

        return carry

    lax.fori_loop(0, (_ROWS + _NW - 1) // _NW, _row, 0)


def kernel(scores, M):
    B, C = scores.shape
    L = M.shape[0]
    sT = jnp.swapaxes(scores, 0, 1)
    mflags = jnp.pad(M.astype(jnp.int32).reshape(L * C), (0, 20))

    mesh = plsc.VectorSubcoreMesh(core_axis_name="c", subcore_axis_name="s")
    k = functools.partial(
        pl.kernel,
        mesh=mesh,
        out_type=jax.ShapeDtypeStruct((L, C, B), jnp.float32),
        scratch_types=[
            pltpu.VMEM((B,), jnp.float32),
            pltpu.VMEM((B,), jnp.float32),
            pltpu.VMEM((L * C + 20,), jnp.int32),
            pltpu.SemaphoreType.DMA,
        ],
    )(_sc_body)
    outT = k(sT, mflags)
    return jnp.transpose(outT, (2, 0, 1))
